# baseline (speedup 1.0000x reference)
import jax
import jax.numpy as jnp
from jax.experimental import pallas as pl
from jax.experimental.pallas import tpu as pltpu

U = 5000
I = 5000
N = U + I
E = 320000
D = 128
T = 50
B = 1024
L = 2


def _fuse_body(sp_ref, tm_ref, w_ref, b_ref, out_ref):
    sp = sp_ref[...]
    tm = tm_ref[...]
    w = w_ref[...]
    b = b_ref[...]
    l0 = (jnp.sum(sp * w[0, :D][None, :], axis=1)
          + jnp.sum(tm * w[0, D:][None, :], axis=1) + b[0, 0])
    l1 = (jnp.sum(sp * w[1, :D][None, :], axis=1)
          + jnp.sum(tm * w[1, D:][None, :], axis=1) + b[0, 1])
    m = jnp.maximum(l0, l1)
    e0 = jnp.exp(l0 - m)
    e1 = jnp.exp(l1 - m)
    s = e0 + e1
    w0 = e0 / s
    w1 = e1 / s
    out_ref[...] = w0[:, None] * sp + w1[:, None] * tm


def _normalize(x):
    n = jnp.sqrt(jnp.sum(x * x, axis=1, keepdims=True))
    return x / jnp.maximum(n, 1e-12)


def _spmm(rows, cols, vals, x):
    msgs = vals[:, None] * jnp.take(x, cols, axis=0)
    return jax.ops.segment_sum(msgs, rows, num_segments=N)


def _lstm_last(seq, Wih, Whh, bih, bhh):
    Bn = seq.shape[0]
    h0 = jnp.zeros((Bn, D), dtype=seq.dtype)
    c0 = jnp.zeros((Bn, D), dtype=seq.dtype)

    def step(carry, x_t):
        h, c = carry
        z = x_t @ Wih.T + h @ Whh.T + bih + bhh
        i, f, g, o = jnp.split(z, 4, axis=1)
        i = jax.nn.sigmoid(i)
        f = jax.nn.sigmoid(f)
        g = jnp.tanh(g)
        o = jax.nn.sigmoid(o)
        c = f * c + i * g
        h = o * jnp.tanh(c)
        return (h, c), None

    (h, c), _ = jax.lax.scan(step, (h0, c0), jnp.transpose(seq, (1, 0, 2)))
    return h


def kernel(adj_indices, adj_vals, user_seq_map, users, user_emb, item_emb,
           Wih, Whh, bih, bhh, attn_W, attn_b):
    rows = adj_indices[0]
    cols = adj_indices[1]
    x = jnp.concatenate([user_emb, item_emb], axis=0)
    final = x
    h = x
    for _ in range(L):
        h = _spmm(rows, cols, adj_vals, h)
        final = final + h
    ue = _normalize(final[:U])
    ie = _normalize(final[U:])
    batch_seq = jnp.take(user_seq_map, users, axis=0)
    seq_emb = jnp.take(ie, batch_seq, axis=0)
    temporal = _lstm_last(seq_emb, Wih, Whh, bih, bhh)
    spatial = jnp.take(ue, users, axis=0)

    fused = pl.pallas_call(
        _fuse_body,
        out_shape=jax.ShapeDtypeStruct((B, D), jnp.float32),
    )(spatial, temporal, attn_W, attn_b.reshape(1, 2))
    return (fused, ie)


# speedup vs baseline: 4.4248x; 4.4248x over previous
import jax
import jax.numpy as jnp
from jax import lax
from jax.experimental import pallas as pl
from jax.experimental.pallas import tpu as pltpu
from jax.experimental.pallas import tpu_sc as plsc

U = 5000
I = 5000
N = U + I
E = 320000
D = 128
T = 50
B = 1024

NC = 2
NS = 16
NW = NC * NS
EPW = 10240
E_PAD = NW * EPW
CH = 128
NCH = EPW // CH
NPAD = 10240
RPT = NPAD // NS
UPW = B // NW
TPADC = 128


def _lane_bcast(v, k):
    idx = jnp.full((16, 1), k, jnp.int32)
    dn = lax.GatherDimensionNumbers(offset_dims=(), collapsed_slice_dims=(0,),
                                    start_index_map=(0,))
    return lax.gather(v, idx, dn, (1,),
                      mode=lax.GatherScatterMode.PROMISE_IN_BOUNDS)


def _spmm_sc(rows_hbm, cols_hbm, vals_hbm, x_hbm, out_hbm,
             acc, colbuf, rowbuf, valbuf, gbuf, gsem):
    c = lax.axis_index("c")
    s = lax.axis_index("s")
    wid = s * NC + c
    base = wid * EPW

    zv = jnp.zeros((16,), jnp.float32)

    def zrow(r, carry):
        for j in range(8):
            gbuf[r, pl.ds(j * 16, 16)] = zv
        return carry

    lax.fori_loop(0, CH, zrow, 0)
    r0 = s * RPT
    for k in range(RPT // CH):
        pltpu.sync_copy(gbuf, acc.at[pl.ds(r0 + k * CH, CH)])
    plsc.subcore_barrier()

    def chunk(i, carry):
        off = base + i * CH
        pltpu.sync_copy(cols_hbm.at[pl.ds(off, CH)], colbuf)
        pltpu.sync_copy(rows_hbm.at[pl.ds(off, CH)], rowbuf)
        pltpu.sync_copy(vals_hbm.at[pl.ds(off, CH)], valbuf)
        pltpu.async_copy(x_hbm.at[colbuf], gbuf, gsem).wait()

        def scale_group(g, carry2):
            vval = valbuf[pl.ds(g * 16, 16)]
            for k in range(16):
                e = g * 16 + k
                vv = _lane_bcast(vval, k)
                for j in range(8):
                    sl = pl.ds(j * 16, 16)
                    gbuf[e, sl] = gbuf[e, sl] * vv
            return carry2

        lax.fori_loop(0, CH // 16, scale_group, 0)
        pltpu.sync_copy(gbuf, acc.at[rowbuf], add=True)
        return carry

    lax.fori_loop(0, NCH, chunk, 0)
    plsc.subcore_barrier()

    for k in range(RPT // CH):
        sl = pl.ds(r0 + k * CH, CH)
        pltpu.sync_copy(acc.at[sl], gbuf)
        pltpu.sync_copy(gbuf, out_hbm.at[c, sl])


def _spmm_call(rows_p, cols_p, vals_p, x):
    mesh = plsc.VectorSubcoreMesh(core_axis_name="c", subcore_axis_name="s")
    fn = pl.kernel(
        _spmm_sc,
        out_type=jax.ShapeDtypeStruct((NC, NPAD, D), jnp.float32),
        mesh=mesh,
        scratch_types=[
            pltpu.VMEM_SHARED((NPAD, D), jnp.float32),
            pltpu.VMEM((CH,), jnp.int32),
            pltpu.VMEM((CH,), jnp.int32),
            pltpu.VMEM((CH,), jnp.float32),
            pltpu.VMEM((CH, D), jnp.float32),
            pltpu.SemaphoreType.DMA,
        ],
    )
    return fn(rows_p, cols_p, vals_p, x)


def _gather_sc(seqmap_hbm, users_hbm, table_hbm, seq_out, sp_out,
               ubuf, idxbuf, gbuf, sbuf, sem):
    c = lax.axis_index("c")
    s = lax.axis_index("s")
    wid = s * NC + c
    ub = wid * UPW
    pltpu.sync_copy(users_hbm.at[pl.ds(ub, UPW)], ubuf)
    pltpu.async_copy(seqmap_hbm.at[ubuf], idxbuf, sem).wait()
    pltpu.async_copy(table_hbm.at[ubuf], sbuf, sem).wait()
    pltpu.sync_copy(sbuf, sp_out.at[pl.ds(ub, UPW)])

    def user_loop(u, carry):
        pltpu.async_copy(table_hbm.at[idxbuf.at[u, pl.ds(0, T)]], gbuf,
                         sem).wait()
        pltpu.sync_copy(gbuf, seq_out.at[ub + u])
        return carry

    lax.fori_loop(0, UPW, user_loop, 0)


def _gather_call(seqmap_pad, users, ue_ie):
    mesh = plsc.VectorSubcoreMesh(core_axis_name="c", subcore_axis_name="s")
    fn = pl.kernel(
        _gather_sc,
        out_type=(
            jax.ShapeDtypeStruct((B, T, D), jnp.float32),
            jax.ShapeDtypeStruct((B, D), jnp.float32),
        ),
        mesh=mesh,
        scratch_types=[
            pltpu.VMEM((UPW,), jnp.int32),
            pltpu.VMEM((UPW, TPADC), jnp.int32),
            pltpu.VMEM((T, D), jnp.float32),
            pltpu.VMEM((UPW, D), jnp.float32),
            pltpu.SemaphoreType.DMA,
        ],
    )
    return fn(seqmap_pad, users, ue_ie)


_CBLK = 1000


def _combine1_body(x_ref, p_ref, h1_ref, f1_ref):
    h1 = p_ref[0] + p_ref[1]
    h1_ref[...] = h1
    f1_ref[...] = x_ref[...] + h1


def _combine1(x, p):
    grid = N // _CBLK
    return pl.pallas_call(
        _combine1_body,
        grid=(grid,),
        in_specs=[
            pl.BlockSpec((_CBLK, D), lambda i: (i, 0)),
            pl.BlockSpec((NC, _CBLK, D), lambda i: (0, i, 0)),
        ],
        out_specs=[
            pl.BlockSpec((_CBLK, D), lambda i: (i, 0)),
            pl.BlockSpec((_CBLK, D), lambda i: (i, 0)),
        ],
        out_shape=[
            jax.ShapeDtypeStruct((N, D), jnp.float32),
            jax.ShapeDtypeStruct((N, D), jnp.float32),
        ],
    )(x, p)


def _combine2_body(f1_ref, q_ref, out_ref):
    f = f1_ref[...] + q_ref[0] + q_ref[1]
    nrm = jnp.sqrt(jnp.sum(f * f, axis=1, keepdims=True))
    out_ref[...] = f / jnp.maximum(nrm, 1e-12)


def _combine2(f1, q):
    grid = N // _CBLK
    return pl.pallas_call(
        _combine2_body,
        grid=(grid,),
        in_specs=[
            pl.BlockSpec((_CBLK, D), lambda i: (i, 0)),
            pl.BlockSpec((NC, _CBLK, D), lambda i: (0, i, 0)),
        ],
        out_specs=pl.BlockSpec((_CBLK, D), lambda i: (i, 0)),
        out_shape=jax.ShapeDtypeStruct((N, D), jnp.float32),
    )(f1, q)


def _lstm_body(seq_ref, sp_ref, w_ref, b_ref, aw_ref, ab_ref, out_ref,
               h_ref, c_ref):
    t = pl.program_id(0)

    @pl.when(t == 0)
    def _():
        h_ref[...] = jnp.zeros_like(h_ref)
        c_ref[...] = jnp.zeros_like(c_ref)

    xt = seq_ref[...]
    h = h_ref[...]
    cc = c_ref[...]
    w = w_ref[...]
    z = (jnp.dot(xt, w[:D], preferred_element_type=jnp.float32)
         + jnp.dot(h, w[D:], preferred_element_type=jnp.float32)
         + b_ref[...])
    ig = jax.nn.sigmoid(z[:, :D])
    fg = jax.nn.sigmoid(z[:, D:2 * D])
    gg = jnp.tanh(z[:, 2 * D:3 * D])
    og = jax.nn.sigmoid(z[:, 3 * D:])
    cc = fg * cc + ig * gg
    h = og * jnp.tanh(cc)
    h_ref[...] = h
    c_ref[...] = cc

    @pl.when(t == T - 1)
    def _():
        sp = sp_ref[...]
        aw = aw_ref[...]
        ab = ab_ref[...]
        l0 = (jnp.sum(sp * aw[0, :D][None, :], axis=1)
              + jnp.sum(h * aw[0, D:][None, :], axis=1) + ab[0, 0])
        l1 = (jnp.sum(sp * aw[1, :D][None, :], axis=1)
              + jnp.sum(h * aw[1, D:][None, :], axis=1) + ab[0, 1])
        m = jnp.maximum(l0, l1)
        e0 = jnp.exp(l0 - m)
        e1 = jnp.exp(l1 - m)
        ssum = e0 + e1
        w0 = e0 / ssum
        w1 = e1 / ssum
        out_ref[...] = w0[:, None] * sp + w1[:, None] * h


def _lstm_call(seq2, spatial, Wc, bcomb, attn_W, attn_b2):
    return pl.pallas_call(
        _lstm_body,
        grid=(T,),
        in_specs=[
            pl.BlockSpec((B, D), lambda t: (0, t)),
            pl.BlockSpec((B, D), lambda t: (0, 0)),
            pl.BlockSpec((2 * D, 4 * D), lambda t: (0, 0)),
            pl.BlockSpec((1, 4 * D), lambda t: (0, 0)),
            pl.BlockSpec((2, 2 * D), lambda t: (0, 0)),
            pl.BlockSpec((1, 2), lambda t: (0, 0)),
        ],
        out_specs=pl.BlockSpec((B, D), lambda t: (0, 0)),
        out_shape=jax.ShapeDtypeStruct((B, D), jnp.float32),
        scratch_shapes=[
            pltpu.VMEM((B, D), jnp.float32),
            pltpu.VMEM((B, D), jnp.float32),
        ],
    )(seq2, spatial, Wc, bcomb, attn_W, attn_b2)


def kernel(adj_indices, adj_vals, user_seq_map, users, user_emb, item_emb,
           Wih, Whh, bih, bhh, attn_W, attn_b):
    rows = adj_indices[0].astype(jnp.int32)
    cols = adj_indices[1].astype(jnp.int32)
    pad = E_PAD - E
    pidx = jnp.arange(pad, dtype=jnp.int32) % N
    rows_p = jnp.concatenate([rows, pidx])
    cols_p = jnp.concatenate([cols, pidx])
    vals_p = jnp.concatenate([adj_vals, jnp.zeros((pad,), jnp.float32)])
    x = jnp.concatenate([user_emb, item_emb], axis=0)

    p = _spmm_call(rows_p, cols_p, vals_p, x)
    h1, f1 = _combine1(x, p)
    q = _spmm_call(rows_p, cols_p, vals_p, h1)
    ue_ie = _combine2(f1, q)
    ie = ue_ie[U:]

    seqmap_pad = jnp.pad(user_seq_map.astype(jnp.int32) + U,
                         ((0, 0), (0, TPADC - T)))
    seq_flat, spatial = _gather_call(seqmap_pad, users.astype(jnp.int32),
                                     ue_ie)
    seq2 = seq_flat.reshape(B, T * D)

    Wc = jnp.concatenate([Wih.T, Whh.T], axis=0)
    bcomb = (bih + bhh).reshape(1, 4 * D)
    fused = _lstm_call(seq2, spatial, Wc, bcomb, attn_W, attn_b.reshape(1, 2))
    return (fused, ie)


# speedup vs baseline: 7.5374x; 1.7034x over previous
import jax
import jax.numpy as jnp
from jax import lax
from jax.experimental import pallas as pl
from jax.experimental.pallas import tpu as pltpu
from jax.experimental.pallas import tpu_sc as plsc

U = 5000
I = 5000
N = U + I
E = 320000
D = 128
T = 50
B = 1024

NC = 2
NS = 16
NW = NC * NS
EPW = 10240
E_PAD = NW * EPW
CH = 128
NCH = EPW // CH
NPAD = 10240
RPT = NPAD // NS
UPW = B // NW
TPADC = 128


def _lane_bcast(v, k):
    idx = jnp.full((16, 1), k, jnp.int32)
    dn = lax.GatherDimensionNumbers(offset_dims=(), collapsed_slice_dims=(0,),
                                    start_index_map=(0,))
    return lax.gather(v, idx, dn, (1,),
                      mode=lax.GatherScatterMode.PROMISE_IN_BOUNDS)


def _spmm_sc(rows_hbm, cols_hbm, vals_hbm, x_hbm, out_hbm,
             acc, cols_all, rowbuf, valbuf,
             gb0, gb1, gs0, gs1, ss0, ss1):
    c = lax.axis_index("c")
    s = lax.axis_index("s")
    wid = s * NC + c
    gbufs = (gb0, gb1)
    gsems = (gs0, gs1)
    ssems = (ss0, ss1)

    cb = wid * NCH
    pltpu.sync_copy(cols_hbm.at[pl.ds(cb, NCH)], cols_all)

    def issue_gather(ch, b):
        pltpu.async_copy(x_hbm.at[cols_all.at[ch]], gbufs[b], gsems[b])

    def wait_gather(b):
        pltpu.make_async_copy(x_hbm.at[cols_all.at[0]], gbufs[b],
                              gsems[b]).wait()

    def issue_scatter(b):
        pltpu.async_copy(gbufs[b], acc.at[rowbuf.at[b]], ssems[b], add=True)

    def wait_scatter(b):
        pltpu.make_async_copy(gbufs[b], acc.at[rowbuf.at[b]],
                              ssems[b]).wait()

    issue_gather(0, 0)

    zv = jnp.zeros((16,), jnp.float32)

    def zrow(r, carry):
        for j in range(8):
            gb1[r, pl.ds(j * 16, 16)] = zv
        return carry

    lax.fori_loop(0, CH, zrow, 0)
    r0 = s * RPT
    for k in range(RPT // CH):
        pltpu.sync_copy(gb1, acc.at[pl.ds(r0 + k * CH, CH)])
    plsc.subcore_barrier()

    def pair(i2, carry):
        for b in range(2):
            ch = i2 * 2 + b
            nb = 1 - b
            pltpu.sync_copy(vals_hbm.at[cb + ch], valbuf.at[b])
            pltpu.sync_copy(rows_hbm.at[cb + ch], rowbuf.at[b])

            @pl.when(ch >= 1)
            def _():
                wait_scatter(nb)

            @pl.when(ch + 1 < NCH)
            def _():
                issue_gather(ch + 1, nb)

            wait_gather(b)

            def scale_group(g, carry2):
                vval = valbuf[b, pl.ds(g * 16, 16)]
                gbuf = gbufs[b]
                for k in range(16):
                    e = g * 16 + k
                    vv = _lane_bcast(vval, k)
                    for j in range(8):
                        sl = pl.ds(j * 16, 16)
                        gbuf[e, sl] = gbuf[e, sl] * vv
                return carry2

            lax.fori_loop(0, CH // 16, scale_group, 0)
            issue_scatter(b)
        return carry

    lax.fori_loop(0, NCH // 2, pair, 0)
    wait_scatter(1)
    plsc.subcore_barrier()

    for k in range(RPT // CH):
        sl = pl.ds(r0 + k * CH, CH)
        pltpu.sync_copy(acc.at[sl], gb0)
        pltpu.sync_copy(gb0, out_hbm.at[c, sl])


def _spmm_call(rows2, cols2, vals2, x):
    mesh = plsc.VectorSubcoreMesh(core_axis_name="c", subcore_axis_name="s")
    fn = pl.kernel(
        _spmm_sc,
        out_type=jax.ShapeDtypeStruct((NC, NPAD, D), jnp.float32),
        mesh=mesh,
        scratch_types=[
            pltpu.VMEM_SHARED((NPAD, D), jnp.float32),
            pltpu.VMEM((NCH, CH), jnp.int32),
            pltpu.VMEM((2, CH), jnp.int32),
            pltpu.VMEM((2, CH), jnp.float32),
            pltpu.VMEM((CH, D), jnp.float32),
            pltpu.VMEM((CH, D), jnp.float32),
            pltpu.SemaphoreType.DMA,
            pltpu.SemaphoreType.DMA,
            pltpu.SemaphoreType.DMA,
            pltpu.SemaphoreType.DMA,
        ],
    )
    return fn(rows2, cols2, vals2, x)


def _gather_sc(seqmap_hbm, users_hbm, table_hbm, seq_out, sp_out,
               ubuf, idxbuf, gbuf, sbuf, sem):
    c = lax.axis_index("c")
    s = lax.axis_index("s")
    wid = s * NC + c
    ub = wid * UPW
    pltpu.sync_copy(users_hbm.at[pl.ds(ub, UPW)], ubuf)
    pltpu.async_copy(seqmap_hbm.at[ubuf], idxbuf, sem).wait()
    pltpu.async_copy(table_hbm.at[ubuf], sbuf, sem).wait()
    pltpu.sync_copy(sbuf, sp_out.at[pl.ds(ub, UPW)])

    def user_loop(u, carry):
        pltpu.async_copy(table_hbm.at[idxbuf.at[u, pl.ds(0, T)]], gbuf,
                         sem).wait()
        pltpu.sync_copy(gbuf, seq_out.at[ub + u])
        return carry

    lax.fori_loop(0, UPW, user_loop, 0)


def _gather_call(seqmap_pad, users, ue_ie):
    mesh = plsc.VectorSubcoreMesh(core_axis_name="c", subcore_axis_name="s")
    fn = pl.kernel(
        _gather_sc,
        out_type=(
            jax.ShapeDtypeStruct((B, T, D), jnp.float32),
            jax.ShapeDtypeStruct((B, D), jnp.float32),
        ),
        mesh=mesh,
        scratch_types=[
            pltpu.VMEM((UPW,), jnp.int32),
            pltpu.VMEM((UPW, TPADC), jnp.int32),
            pltpu.VMEM((T, D), jnp.float32),
            pltpu.VMEM((UPW, D), jnp.float32),
            pltpu.SemaphoreType.DMA,
        ],
    )
    return fn(seqmap_pad, users, ue_ie)


_CBLK = 1000


def _combine1_body(x_ref, p_ref, h1_ref, f1_ref):
    h1 = p_ref[0] + p_ref[1]
    h1_ref[...] = h1
    f1_ref[...] = x_ref[...] + h1


def _combine1(x, p):
    grid = N // _CBLK
    return pl.pallas_call(
        _combine1_body,
        grid=(grid,),
        in_specs=[
            pl.BlockSpec((_CBLK, D), lambda i: (i, 0)),
            pl.BlockSpec((NC, _CBLK, D), lambda i: (0, i, 0)),
        ],
        out_specs=[
            pl.BlockSpec((_CBLK, D), lambda i: (i, 0)),
            pl.BlockSpec((_CBLK, D), lambda i: (i, 0)),
        ],
        out_shape=[
            jax.ShapeDtypeStruct((N, D), jnp.float32),
            jax.ShapeDtypeStruct((N, D), jnp.float32),
        ],
    )(x, p)


def _combine2_body(f1_ref, q_ref, out_ref):
    f = f1_ref[...] + q_ref[0] + q_ref[1]
    nrm = jnp.sqrt(jnp.sum(f * f, axis=1, keepdims=True))
    out_ref[...] = f / jnp.maximum(nrm, 1e-12)


def _combine2(f1, q):
    grid = N // _CBLK
    return pl.pallas_call(
        _combine2_body,
        grid=(grid,),
        in_specs=[
            pl.BlockSpec((_CBLK, D), lambda i: (i, 0)),
            pl.BlockSpec((NC, _CBLK, D), lambda i: (0, i, 0)),
        ],
        out_specs=pl.BlockSpec((_CBLK, D), lambda i: (i, 0)),
        out_shape=jax.ShapeDtypeStruct((N, D), jnp.float32),
    )(f1, q)


def _lstm_body(seq_ref, sp_ref, w_ref, b_ref, aw_ref, ab_ref, out_ref,
               h_ref, c_ref):
    t = pl.program_id(0)

    @pl.when(t == 0)
    def _():
        h_ref[...] = jnp.zeros_like(h_ref)
        c_ref[...] = jnp.zeros_like(c_ref)

    xt = seq_ref[...]
    h = h_ref[...]
    cc = c_ref[...]
    w = w_ref[...]
    z = (jnp.dot(xt, w[:D], preferred_element_type=jnp.float32)
         + jnp.dot(h, w[D:], preferred_element_type=jnp.float32)
         + b_ref[...])
    ig = jax.nn.sigmoid(z[:, :D])
    fg = jax.nn.sigmoid(z[:, D:2 * D])
    gg = jnp.tanh(z[:, 2 * D:3 * D])
    og = jax.nn.sigmoid(z[:, 3 * D:])
    cc = fg * cc + ig * gg
    h = og * jnp.tanh(cc)
    h_ref[...] = h
    c_ref[...] = cc

    @pl.when(t == T - 1)
    def _():
        sp = sp_ref[...]
        aw = aw_ref[...]
        ab = ab_ref[...]
        l0 = (jnp.sum(sp * aw[0, :D][None, :], axis=1)
              + jnp.sum(h * aw[0, D:][None, :], axis=1) + ab[0, 0])
        l1 = (jnp.sum(sp * aw[1, :D][None, :], axis=1)
              + jnp.sum(h * aw[1, D:][None, :], axis=1) + ab[0, 1])
        m = jnp.maximum(l0, l1)
        e0 = jnp.exp(l0 - m)
        e1 = jnp.exp(l1 - m)
        ssum = e0 + e1
        w0 = e0 / ssum
        w1 = e1 / ssum
        out_ref[...] = w0[:, None] * sp + w1[:, None] * h


def _lstm_call(seq2, spatial, Wc, bcomb, attn_W, attn_b2):
    return pl.pallas_call(
        _lstm_body,
        grid=(T,),
        in_specs=[
            pl.BlockSpec((B, D), lambda t: (0, t)),
            pl.BlockSpec((B, D), lambda t: (0, 0)),
            pl.BlockSpec((2 * D, 4 * D), lambda t: (0, 0)),
            pl.BlockSpec((1, 4 * D), lambda t: (0, 0)),
            pl.BlockSpec((2, 2 * D), lambda t: (0, 0)),
            pl.BlockSpec((1, 2), lambda t: (0, 0)),
        ],
        out_specs=pl.BlockSpec((B, D), lambda t: (0, 0)),
        out_shape=jax.ShapeDtypeStruct((B, D), jnp.float32),
        scratch_shapes=[
            pltpu.VMEM((B, D), jnp.float32),
            pltpu.VMEM((B, D), jnp.float32),
        ],
    )(seq2, spatial, Wc, bcomb, attn_W, attn_b2)


def kernel(adj_indices, adj_vals, user_seq_map, users, user_emb, item_emb,
           Wih, Whh, bih, bhh, attn_W, attn_b):
    rows = adj_indices[0].astype(jnp.int32)
    cols = adj_indices[1].astype(jnp.int32)
    pad = E_PAD - E
    pidx = jnp.arange(pad, dtype=jnp.int32) % N
    rows_p = jnp.concatenate([rows, pidx]).reshape(E_PAD // CH, CH)
    cols_p = jnp.concatenate([cols, pidx]).reshape(E_PAD // CH, CH)
    vals_p = jnp.concatenate([adj_vals, jnp.zeros((pad,), jnp.float32)]
                             ).reshape(E_PAD // CH, CH)
    x = jnp.concatenate([user_emb, item_emb], axis=0)

    p = _spmm_call(rows_p, cols_p, vals_p, x)
    h1, f1 = _combine1(x, p)
    q = _spmm_call(rows_p, cols_p, vals_p, h1)
    ue_ie = _combine2(f1, q)
    ie = ue_ie[U:]

    seqmap_pad = jnp.pad(user_seq_map.astype(jnp.int32) + U,
                         ((0, 0), (0, TPADC - T)))
    seq_flat, spatial = _gather_call(seqmap_pad, users.astype(jnp.int32),
                                     ue_ie)
    seq2 = seq_flat.reshape(B, T * D)

    Wc = jnp.concatenate([Wih.T, Whh.T], axis=0)
    bcomb = (bih + bhh).reshape(1, 4 * D)
    fused = _lstm_call(seq2, spatial, Wc, bcomb, attn_W, attn_b.reshape(1, 2))
    return (fused, ie)


# speedup vs baseline: 7.8056x; 1.0356x over previous
import jax
import jax.numpy as jnp
from jax import lax
from jax.experimental import pallas as pl
from jax.experimental.pallas import tpu as pltpu
from jax.experimental.pallas import tpu_sc as plsc

U = 5000
I = 5000
N = U + I
E = 320000
D = 128
T = 50
B = 1024

NC = 2
NS = 16
NW = NC * NS
EPW = 10240
E_PAD = NW * EPW
CH = 128
NCH = EPW // CH
NPAD = 10240
RPT = NPAD // NS
UPW = B // NW
TPADC = 128


def _lane_bcast(v, k):
    idx = jnp.full((16, 1), k, jnp.int32)
    dn = lax.GatherDimensionNumbers(offset_dims=(), collapsed_slice_dims=(0,),
                                    start_index_map=(0,))
    return lax.gather(v, idx, dn, (1,),
                      mode=lax.GatherScatterMode.PROMISE_IN_BOUNDS)


def _spmm_sc(rows_hbm, cols_hbm, vals_hbm, x_hbm, out_hbm,
             acc, cols_all, rowbuf, valbuf,
             gb0, gb1, gs0, gs1, ss0, ss1):
    c = lax.axis_index("c")
    s = lax.axis_index("s")
    wid = s * NC + c
    gbufs = (gb0, gb1)
    gsems = (gs0, gs1)
    ssems = (ss0, ss1)

    cb = wid * NCH
    pltpu.sync_copy(cols_hbm.at[pl.ds(cb, NCH)], cols_all)

    def issue_gather(ch, b):
        pltpu.async_copy(x_hbm.at[cols_all.at[ch]], gbufs[b], gsems[b])

    def wait_gather(b):
        pltpu.make_async_copy(x_hbm.at[cols_all.at[0]], gbufs[b],
                              gsems[b]).wait()

    def issue_scatter(b):
        pltpu.async_copy(gbufs[b], acc.at[rowbuf.at[b]], ssems[b], add=True)

    def wait_scatter(b):
        pltpu.make_async_copy(gbufs[b], acc.at[rowbuf.at[b]],
                              ssems[b]).wait()

    issue_gather(0, 0)

    zv = jnp.zeros((16,), jnp.float32)

    def zrow(r, carry):
        for j in range(8):
            gb1[r, pl.ds(j * 16, 16)] = zv
        return carry

    lax.fori_loop(0, CH, zrow, 0)
    r0 = s * RPT
    for k in range(RPT // CH):
        pltpu.sync_copy(gb1, acc.at[pl.ds(r0 + k * CH, CH)])
    plsc.subcore_barrier()

    def pair(i2, carry):
        for b in range(2):
            ch = i2 * 2 + b
            nb = 1 - b
            pltpu.sync_copy(vals_hbm.at[cb + ch], valbuf.at[b])
            pltpu.sync_copy(rows_hbm.at[cb + ch], rowbuf.at[b])

            @pl.when(ch >= 1)
            def _():
                wait_scatter(nb)

            @pl.when(ch + 1 < NCH)
            def _():
                issue_gather(ch + 1, nb)

            wait_gather(b)

            def scale_group(g, carry2):
                vval = valbuf[b, pl.ds(g * 16, 16)]
                gbuf = gbufs[b]
                for k in range(16):
                    e = g * 16 + k
                    vv = _lane_bcast(vval, k)
                    for j in range(8):
                        sl = pl.ds(j * 16, 16)
                        gbuf[e, sl] = gbuf[e, sl] * vv
                return carry2

            lax.fori_loop(0, CH // 16, scale_group, 0)
            issue_scatter(b)
        return carry

    lax.fori_loop(0, NCH // 2, pair, 0)
    wait_scatter(1)
    plsc.subcore_barrier()

    for k in range(RPT // CH):
        sl = pl.ds(r0 + k * CH, CH)
        pltpu.sync_copy(acc.at[sl], gb0)
        pltpu.sync_copy(gb0, out_hbm.at[c, sl])


def _spmm_call(rows2, cols2, vals2, x):
    mesh = plsc.VectorSubcoreMesh(core_axis_name="c", subcore_axis_name="s")
    fn = pl.kernel(
        _spmm_sc,
        out_type=jax.ShapeDtypeStruct((NC, NPAD, D), jnp.float32),
        mesh=mesh,
        scratch_types=[
            pltpu.VMEM_SHARED((NPAD, D), jnp.float32),
            pltpu.VMEM((NCH, CH), jnp.int32),
            pltpu.VMEM((2, CH), jnp.int32),
            pltpu.VMEM((2, CH), jnp.float32),
            pltpu.VMEM((CH, D), jnp.float32),
            pltpu.VMEM((CH, D), jnp.float32),
            pltpu.SemaphoreType.DMA,
            pltpu.SemaphoreType.DMA,
            pltpu.SemaphoreType.DMA,
            pltpu.SemaphoreType.DMA,
        ],
    )
    return fn(rows2, cols2, vals2, x)


def _gather_sc(seqmap_hbm, users_hbm, table_hbm, seq_out, sp_out,
               ubuf, idxbuf, gb0, gb1, sbuf, sem, gs0, gs1):
    c = lax.axis_index("c")
    s = lax.axis_index("s")
    wid = s * NC + c
    ub = wid * UPW
    gbufs = (gb0, gb1)
    gsems = (gs0, gs1)
    pltpu.sync_copy(users_hbm.at[pl.ds(ub, UPW)], ubuf)
    pltpu.async_copy(seqmap_hbm.at[ubuf], idxbuf, sem).wait()

    def issue_user(u, b):
        pltpu.async_copy(table_hbm.at[idxbuf.at[u, pl.ds(0, T)]], gbufs[b],
                         gsems[b])

    def wait_user(b):
        pltpu.make_async_copy(table_hbm.at[idxbuf.at[0, pl.ds(0, T)]],
                              gbufs[b], gsems[b]).wait()

    issue_user(0, 0)
    pltpu.async_copy(table_hbm.at[ubuf], sbuf, sem).wait()
    pltpu.sync_copy(sbuf, sp_out.at[pl.ds(ub, UPW)])

    def pair(i2, carry):
        for b in range(2):
            u = i2 * 2 + b

            @pl.when(u + 1 < UPW)
            def _():
                issue_user(u + 1, 1 - b)

            wait_user(b)
            pltpu.sync_copy(gbufs[b], seq_out.at[ub + u])
        return carry

    lax.fori_loop(0, UPW // 2, pair, 0)


def _gather_call(seqmap_pad, users, ue_ie):
    mesh = plsc.VectorSubcoreMesh(core_axis_name="c", subcore_axis_name="s")
    fn = pl.kernel(
        _gather_sc,
        out_type=(
            jax.ShapeDtypeStruct((B, T, D), jnp.float32),
            jax.ShapeDtypeStruct((B, D), jnp.float32),
        ),
        mesh=mesh,
        scratch_types=[
            pltpu.VMEM((UPW,), jnp.int32),
            pltpu.VMEM((UPW, TPADC), jnp.int32),
            pltpu.VMEM((T, D), jnp.float32),
            pltpu.VMEM((T, D), jnp.float32),
            pltpu.VMEM((UPW, D), jnp.float32),
            pltpu.SemaphoreType.DMA,
            pltpu.SemaphoreType.DMA,
            pltpu.SemaphoreType.DMA,
        ],
    )
    return fn(seqmap_pad, users, ue_ie)


_CBLK = 1000


def _combine1_body(x_ref, p_ref, h1_ref, f1_ref):
    h1 = p_ref[0] + p_ref[1]
    h1_ref[...] = h1
    f1_ref[...] = x_ref[...] + h1


def _combine1(x, p):
    grid = N // _CBLK
    return pl.pallas_call(
        _combine1_body,
        grid=(grid,),
        in_specs=[
            pl.BlockSpec((_CBLK, D), lambda i: (i, 0)),
            pl.BlockSpec((NC, _CBLK, D), lambda i: (0, i, 0)),
        ],
        out_specs=[
            pl.BlockSpec((_CBLK, D), lambda i: (i, 0)),
            pl.BlockSpec((_CBLK, D), lambda i: (i, 0)),
        ],
        out_shape=[
            jax.ShapeDtypeStruct((N, D), jnp.float32),
            jax.ShapeDtypeStruct((N, D), jnp.float32),
        ],
    )(x, p)


def _combine2_body(f1_ref, q_ref, out_ref):
    f = f1_ref[...] + q_ref[0] + q_ref[1]
    nrm = jnp.sqrt(jnp.sum(f * f, axis=1, keepdims=True))
    out_ref[...] = f / jnp.maximum(nrm, 1e-12)


def _combine2(f1, q):
    grid = N // _CBLK
    return pl.pallas_call(
        _combine2_body,
        grid=(grid,),
        in_specs=[
            pl.BlockSpec((_CBLK, D), lambda i: (i, 0)),
            pl.BlockSpec((NC, _CBLK, D), lambda i: (0, i, 0)),
        ],
        out_specs=pl.BlockSpec((_CBLK, D), lambda i: (i, 0)),
        out_shape=jax.ShapeDtypeStruct((N, D), jnp.float32),
    )(f1, q)


def _lstm_body(seq_ref, sp_ref, w_ref, b_ref, aw_ref, ab_ref, out_ref,
               h_ref, c_ref):
    t = pl.program_id(0)

    @pl.when(t == 0)
    def _():
        h_ref[...] = jnp.zeros_like(h_ref)
        c_ref[...] = jnp.zeros_like(c_ref)

    xt = seq_ref[...]
    h = h_ref[...]
    cc = c_ref[...]
    w = w_ref[...]
    z = (jnp.dot(xt.astype(jnp.bfloat16), w[:D],
                 preferred_element_type=jnp.float32)
         + jnp.dot(h.astype(jnp.bfloat16), w[D:],
                   preferred_element_type=jnp.float32)
         + b_ref[...])
    ig = jax.nn.sigmoid(z[:, :D])
    fg = jax.nn.sigmoid(z[:, D:2 * D])
    gg = jnp.tanh(z[:, 2 * D:3 * D])
    og = jax.nn.sigmoid(z[:, 3 * D:])
    cc = fg * cc + ig * gg
    h = og * jnp.tanh(cc)
    h_ref[...] = h
    c_ref[...] = cc

    @pl.when(t == T - 1)
    def _():
        sp = sp_ref[...]
        aw = aw_ref[...]
        ab = ab_ref[...]
        l0 = (jnp.sum(sp * aw[0, :D][None, :], axis=1)
              + jnp.sum(h * aw[0, D:][None, :], axis=1) + ab[0, 0])
        l1 = (jnp.sum(sp * aw[1, :D][None, :], axis=1)
              + jnp.sum(h * aw[1, D:][None, :], axis=1) + ab[0, 1])
        m = jnp.maximum(l0, l1)
        e0 = jnp.exp(l0 - m)
        e1 = jnp.exp(l1 - m)
        ssum = e0 + e1
        w0 = e0 / ssum
        w1 = e1 / ssum
        out_ref[...] = w0[:, None] * sp + w1[:, None] * h


def _lstm_call(seq2, spatial, Wc, bcomb, attn_W, attn_b2):
    return pl.pallas_call(
        _lstm_body,
        grid=(T,),
        in_specs=[
            pl.BlockSpec((B, D), lambda t: (0, t)),
            pl.BlockSpec((B, D), lambda t: (0, 0)),
            pl.BlockSpec((2 * D, 4 * D), lambda t: (0, 0)),
            pl.BlockSpec((1, 4 * D), lambda t: (0, 0)),
            pl.BlockSpec((2, 2 * D), lambda t: (0, 0)),
            pl.BlockSpec((1, 2), lambda t: (0, 0)),
        ],
        out_specs=pl.BlockSpec((B, D), lambda t: (0, 0)),
        out_shape=jax.ShapeDtypeStruct((B, D), jnp.float32),
        scratch_shapes=[
            pltpu.VMEM((B, D), jnp.float32),
            pltpu.VMEM((B, D), jnp.float32),
        ],
    )(seq2, spatial, Wc, bcomb, attn_W, attn_b2)


def kernel(adj_indices, adj_vals, user_seq_map, users, user_emb, item_emb,
           Wih, Whh, bih, bhh, attn_W, attn_b):
    rows = adj_indices[0].astype(jnp.int32)
    cols = adj_indices[1].astype(jnp.int32)
    pad = E_PAD - E
    pidx = jnp.arange(pad, dtype=jnp.int32) % N
    rows_p = jnp.concatenate([rows, pidx]).reshape(E_PAD // CH, CH)
    cols_p = jnp.concatenate([cols, pidx]).reshape(E_PAD // CH, CH)
    vals_p = jnp.concatenate([adj_vals, jnp.zeros((pad,), jnp.float32)]
                             ).reshape(E_PAD // CH, CH)
    x = jnp.concatenate([user_emb, item_emb], axis=0)

    p = _spmm_call(rows_p, cols_p, vals_p, x)
    h1, f1 = _combine1(x, p)
    q = _spmm_call(rows_p, cols_p, vals_p, h1)
    ue_ie = _combine2(f1, q)
    ie = ue_ie[U:]

    seqmap_pad = jnp.pad(user_seq_map.astype(jnp.int32) + U,
                         ((0, 0), (0, TPADC - T)))
    seq_flat, spatial = _gather_call(seqmap_pad, users.astype(jnp.int32),
                                     ue_ie)
    seq2 = seq_flat.reshape(B, T * D)

    Wc = jnp.concatenate([Wih.T, Whh.T], axis=0).astype(jnp.bfloat16)
    bcomb = (bih + bhh).reshape(1, 4 * D)
    fused = _lstm_call(seq2, spatial, Wc, bcomb, attn_W, attn_b.reshape(1, 2))
    return (fused, ie)


# speedup vs baseline: 8.5012x; 1.0891x over previous
import jax
import jax.numpy as jnp
from jax import lax
from jax.experimental import pallas as pl
from jax.experimental.pallas import tpu as pltpu
from jax.experimental.pallas import tpu_sc as plsc

U = 5000
I = 5000
N = U + I
E = 320000
D = 128
T = 50
B = 1024

NC = 2
NS = 16
NW = NC * NS
EPW = 10240
E_PAD = NW * EPW
CH = 128
NCH = EPW // CH
NPAD = 10240
RPT = NPAD // NS
UPW = B // NW
TPADC = 128


def _lane_bcast(v, k):
    idx = jnp.full((16, 1), k, jnp.int32)
    dn = lax.GatherDimensionNumbers(offset_dims=(), collapsed_slice_dims=(0,),
                                    start_index_map=(0,))
    return lax.gather(v, idx, dn, (1,),
                      mode=lax.GatherScatterMode.PROMISE_IN_BOUNDS)


def _spmm_sc(rows_hbm, vals_hbm, cols_hbm, x_hbm, out_hbm,
             acc, cols_all, rowbuf, valbuf,
             gb0, gb1, gs0, gs1, ss0, ss1, rs0, rs1, vs0, vs1):
    c = lax.axis_index("c")
    s = lax.axis_index("s")
    wid = s * NC + c
    gbufs = (gb0, gb1)
    gsems = (gs0, gs1)
    ssems = (ss0, ss1)
    rsems = (rs0, rs1)
    vsems = (vs0, vs1)

    cb = wid * NCH
    pltpu.sync_copy(cols_hbm.at[pl.ds(cb, NCH)], cols_all)

    def issue_gather(ch, b):
        pltpu.async_copy(x_hbm.at[cols_all.at[ch]], gbufs[b], gsems[b])

    def wait_gather(b):
        pltpu.make_async_copy(x_hbm.at[cols_all.at[0]], gbufs[b],
                              gsems[b]).wait()

    def issue_rv(ch, b):
        pltpu.async_copy(rows_hbm.at[cb + ch], rowbuf.at[b], rsems[b])
        pltpu.async_copy(vals_hbm.at[cb + ch], valbuf.at[b], vsems[b])

    def wait_rv(b):
        pltpu.make_async_copy(rows_hbm.at[cb], rowbuf.at[b],
                              rsems[b]).wait()
        pltpu.make_async_copy(vals_hbm.at[cb], valbuf.at[b],
                              vsems[b]).wait()

    def issue_scatter(b):
        pltpu.async_copy(gbufs[b], acc.at[rowbuf.at[b]], ssems[b],
                         add=True)

    def wait_scatter(b):
        pltpu.make_async_copy(gbufs[b], acc.at[rowbuf.at[b]],
                              ssems[b]).wait()

    issue_rv(0, 0)
    issue_gather(0, 0)

    zv = jnp.zeros((16,), jnp.float32)

    def zrow(r, carry):
        for j in range(8):
            gb1[r, pl.ds(j * 16, 16)] = zv
        return carry

    lax.fori_loop(0, CH, zrow, 0)
    r0 = s * RPT
    for k in range(RPT // CH):
        pltpu.sync_copy(gb1, acc.at[pl.ds(r0 + k * CH, CH)])
    plsc.subcore_barrier()

    def pair(i2, carry):
        for b in range(2):
            ch = i2 * 2 + b
            nb = 1 - b
            wait_rv(b)

            @pl.when(ch >= 1)
            def _():
                wait_scatter(nb)

            @pl.when(ch + 1 < NCH)
            def _():
                issue_rv(ch + 1, nb)
                issue_gather(ch + 1, nb)

            wait_gather(b)

            def scale_group(g, carry2):
                vval = valbuf[b, pl.ds(g * 16, 16)]
                gbuf = gbufs[b]
                for k in range(16):
                    e = g * 16 + k
                    vv = _lane_bcast(vval, k)
                    for j in range(8):
                        sl = pl.ds(j * 16, 16)
                        gbuf[e, sl] = gbuf[e, sl] * vv
                return carry2

            lax.fori_loop(0, CH // 16, scale_group, 0)
            issue_scatter(b)
        return carry

    lax.fori_loop(0, NCH // 2, pair, 0)
    wait_scatter(1)
    plsc.subcore_barrier()

    for k in range(RPT // CH):
        sl = pl.ds(r0 + k * CH, CH)
        pltpu.sync_copy(acc.at[sl], gb0)
        pltpu.sync_copy(gb0, out_hbm.at[c, sl])


def _spmm_call(rows2, vals2, cols2, x):
    mesh = plsc.VectorSubcoreMesh(core_axis_name="c", subcore_axis_name="s")
    fn = pl.kernel(
        _spmm_sc,
        out_type=jax.ShapeDtypeStruct((NC, NPAD, D), jnp.float32),
        mesh=mesh,
        scratch_types=[
            pltpu.VMEM_SHARED((NPAD, D), jnp.float32),
            pltpu.VMEM((NCH, CH), jnp.int32),
            pltpu.VMEM((2, CH), jnp.int32),
            pltpu.VMEM((2, CH), jnp.float32),
            pltpu.VMEM((CH, D), jnp.float32),
            pltpu.VMEM((CH, D), jnp.float32),
            pltpu.SemaphoreType.DMA,
            pltpu.SemaphoreType.DMA,
            pltpu.SemaphoreType.DMA,
            pltpu.SemaphoreType.DMA,
            pltpu.SemaphoreType.DMA,
            pltpu.SemaphoreType.DMA,
            pltpu.SemaphoreType.DMA,
            pltpu.SemaphoreType.DMA,
        ],
    )
    return fn(rows2, vals2, cols2, x)


def _gather_sc(seqmap_hbm, users_hbm, table_hbm, seq_out, sp_out,
               ubuf, idxbuf, gb0, gb1, sbuf, sem, gs0, gs1):
    c = lax.axis_index("c")
    s = lax.axis_index("s")
    wid = s * NC + c
    ub = wid * UPW
    gbufs = (gb0, gb1)
    gsems = (gs0, gs1)
    pltpu.sync_copy(users_hbm.at[pl.ds(ub, UPW)], ubuf)
    pltpu.async_copy(seqmap_hbm.at[ubuf], idxbuf, sem).wait()

    def issue_user(u, b):
        pltpu.async_copy(table_hbm.at[idxbuf.at[u, pl.ds(0, T)]], gbufs[b],
                         gsems[b])

    def wait_user(b):
        pltpu.make_async_copy(table_hbm.at[idxbuf.at[0, pl.ds(0, T)]],
                              gbufs[b], gsems[b]).wait()

    issue_user(0, 0)
    pltpu.async_copy(table_hbm.at[ubuf], sbuf, sem).wait()
    pltpu.sync_copy(sbuf, sp_out.at[pl.ds(ub, UPW)])

    def pair(i2, carry):
        for b in range(2):
            u = i2 * 2 + b

            @pl.when(u + 1 < UPW)
            def _():
                issue_user(u + 1, 1 - b)

            wait_user(b)
            pltpu.sync_copy(gbufs[b], seq_out.at[ub + u])
        return carry

    lax.fori_loop(0, UPW // 2, pair, 0)


def _gather_call(seqmap_pad, users, ue_ie):
    mesh = plsc.VectorSubcoreMesh(core_axis_name="c", subcore_axis_name="s")
    fn = pl.kernel(
        _gather_sc,
        out_type=(
            jax.ShapeDtypeStruct((B, T, D), jnp.float32),
            jax.ShapeDtypeStruct((B, D), jnp.float32),
        ),
        mesh=mesh,
        scratch_types=[
            pltpu.VMEM((UPW,), jnp.int32),
            pltpu.VMEM((UPW, TPADC), jnp.int32),
            pltpu.VMEM((T, D), jnp.float32),
            pltpu.VMEM((T, D), jnp.float32),
            pltpu.VMEM((UPW, D), jnp.float32),
            pltpu.SemaphoreType.DMA,
            pltpu.SemaphoreType.DMA,
            pltpu.SemaphoreType.DMA,
        ],
    )
    return fn(seqmap_pad, users, ue_ie)


_CBLK = 1000


def _combine1_body(x_ref, p_ref, h1_ref, f1_ref):
    h1 = p_ref[0] + p_ref[1]
    h1_ref[...] = h1
    f1_ref[...] = x_ref[...] + h1


def _combine1(x, p):
    grid = N // _CBLK
    return pl.pallas_call(
        _combine1_body,
        grid=(grid,),
        in_specs=[
            pl.BlockSpec((_CBLK, D), lambda i: (i, 0)),
            pl.BlockSpec((NC, _CBLK, D), lambda i: (0, i, 0)),
        ],
        out_specs=[
            pl.BlockSpec((_CBLK, D), lambda i: (i, 0)),
            pl.BlockSpec((_CBLK, D), lambda i: (i, 0)),
        ],
        out_shape=[
            jax.ShapeDtypeStruct((N, D), jnp.float32),
            jax.ShapeDtypeStruct((N, D), jnp.float32),
        ],
    )(x, p)


def _combine2_body(f1_ref, q_ref, out_ref):
    f = f1_ref[...] + q_ref[0] + q_ref[1]
    nrm = jnp.sqrt(jnp.sum(f * f, axis=1, keepdims=True))
    out_ref[...] = f / jnp.maximum(nrm, 1e-12)


def _combine2(f1, q):
    grid = N // _CBLK
    return pl.pallas_call(
        _combine2_body,
        grid=(grid,),
        in_specs=[
            pl.BlockSpec((_CBLK, D), lambda i: (i, 0)),
            pl.BlockSpec((NC, _CBLK, D), lambda i: (0, i, 0)),
        ],
        out_specs=pl.BlockSpec((_CBLK, D), lambda i: (i, 0)),
        out_shape=jax.ShapeDtypeStruct((N, D), jnp.float32),
    )(f1, q)


def _lstm_body(seq_ref, w_ref, b_ref, out_ref, h_ref, c_ref):
    t = pl.program_id(0)

    @pl.when(t == 0)
    def _():
        h_ref[...] = jnp.zeros_like(h_ref)
        c_ref[...] = jnp.zeros_like(c_ref)

    xt = seq_ref[...]
    h = h_ref[...]
    cc = c_ref[...]
    w = w_ref[...]
    z = (jnp.dot(xt.astype(jnp.bfloat16), w[:D],
                 preferred_element_type=jnp.float32)
         + jnp.dot(h.astype(jnp.bfloat16), w[D:],
                   preferred_element_type=jnp.float32)
         + b_ref[...])
    ig = jax.nn.sigmoid(z[:, :D])
    fg = jax.nn.sigmoid(z[:, D:2 * D])
    gg = jnp.tanh(z[:, 2 * D:3 * D])
    og = jax.nn.sigmoid(z[:, 3 * D:])
    cc = fg * cc + ig * gg
    h = og * jnp.tanh(cc)
    h_ref[...] = h
    c_ref[...] = cc

    @pl.when(t == T - 1)
    def _():
        out_ref[...] = h


def _lstm_call(seq2, Wc, bcomb):
    return pl.pallas_call(
        _lstm_body,
        grid=(T,),
        in_specs=[
            pl.BlockSpec((B, D), lambda t: (0, t)),
            pl.BlockSpec((2 * D, 4 * D), lambda t: (0, 0)),
            pl.BlockSpec((1, 4 * D), lambda t: (0, 0)),
        ],
        out_specs=pl.BlockSpec((B, D), lambda t: (0, 0)),
        out_shape=jax.ShapeDtypeStruct((B, D), jnp.float32),
        scratch_shapes=[
            pltpu.VMEM((B, D), jnp.float32),
            pltpu.VMEM((B, D), jnp.float32),
        ],
    )(seq2, Wc, bcomb)


def _fuse_body(sp_ref, tm_ref, aw_ref, ab_ref, out_ref):
    sp = sp_ref[...]
    tm = tm_ref[...]
    aw = aw_ref[...]
    ab = ab_ref[...]
    l0 = (jnp.sum(sp * aw[0, :D][None, :], axis=1)
          + jnp.sum(tm * aw[0, D:][None, :], axis=1) + ab[0, 0])
    l1 = (jnp.sum(sp * aw[1, :D][None, :], axis=1)
          + jnp.sum(tm * aw[1, D:][None, :], axis=1) + ab[0, 1])
    m = jnp.maximum(l0, l1)
    e0 = jnp.exp(l0 - m)
    e1 = jnp.exp(l1 - m)
    ssum = e0 + e1
    w0 = e0 / ssum
    w1 = e1 / ssum
    out_ref[...] = w0[:, None] * sp + w1[:, None] * tm


def _fuse_call(spatial, temporal, attn_W, attn_b2):
    return pl.pallas_call(
        _fuse_body,
        out_shape=jax.ShapeDtypeStruct((B, D), jnp.float32),
    )(spatial, temporal, attn_W, attn_b2)


def kernel(adj_indices, adj_vals, user_seq_map, users, user_emb, item_emb,
           Wih, Whh, bih, bhh, attn_W, attn_b):
    rows = adj_indices[0].astype(jnp.int32)
    cols = adj_indices[1].astype(jnp.int32)
    pad = E_PAD - E
    pidx = jnp.arange(pad, dtype=jnp.int32) % N
    rows_p = jnp.concatenate([rows, pidx]).reshape(E_PAD // CH, CH)
    cols_p = jnp.concatenate([cols, pidx]).reshape(E_PAD // CH, CH)
    vals_p = jnp.concatenate([adj_vals, jnp.zeros((pad,), jnp.float32)]
                             ).reshape(E_PAD // CH, CH)
    x = jnp.concatenate([user_emb, item_emb], axis=0)

    p = _spmm_call(rows_p, vals_p, cols_p, x)
    h1, f1 = _combine1(x, p)
    q = _spmm_call(rows_p, vals_p, cols_p, h1)
    ue_ie = _combine2(f1, q)
    ie = ue_ie[U:]

    seqmap_pad = jnp.pad(user_seq_map.astype(jnp.int32) + U,
                         ((0, 0), (0, TPADC - T)))
    seq_flat, spatial = _gather_call(seqmap_pad, users.astype(jnp.int32),
                                     ue_ie)
    seq2 = seq_flat.reshape(B, T * D)

    Wc = jnp.concatenate([Wih.T, Whh.T], axis=0).astype(jnp.bfloat16)
    bcomb = (bih + bhh).reshape(1, 4 * D)
    temporal = _lstm_call(seq2, Wc, bcomb)
    fused = _fuse_call(spatial, temporal, attn_W, attn_b.reshape(1, 2))
    return (fused, ie)


# speedup vs baseline: 8.5144x; 1.0016x over previous
import jax
import jax.numpy as jnp
from jax import lax
from jax.experimental import pallas as pl
from jax.experimental.pallas import tpu as pltpu
from jax.experimental.pallas import tpu_sc as plsc

U = 5000
I = 5000
N = U + I
E = 320000
D = 128
T = 50
B = 1024

NC = 2
NS = 16
NW = NC * NS
EPW = 10240
E_PAD = NW * EPW
CH = 128
NCH = EPW // CH
NPAD = 10240
RPT = NPAD // NS
UPW = B // NW
TPADC = 128


def _lane_bcast(v, k):
    idx = jnp.full((16, 1), k, jnp.int32)
    dn = lax.GatherDimensionNumbers(offset_dims=(), collapsed_slice_dims=(0,),
                                    start_index_map=(0,))
    return lax.gather(v, idx, dn, (1,),
                      mode=lax.GatherScatterMode.PROMISE_IN_BOUNDS)


def _spmm_sc(rows_hbm, vals_hbm, cols_hbm, x_hbm, out_hbm,
             acc, cols_all, rowbuf, valbuf,
             gb0, gb1, gs0, gs1, ss0, ss1, rs0, rs1, vs0, vs1):
    c = lax.axis_index("c")
    s = lax.axis_index("s")
    wid = s * NC + c
    gbufs = (gb0, gb1)
    gsems = (gs0, gs1)
    ssems = (ss0, ss1)
    rsems = (rs0, rs1)
    vsems = (vs0, vs1)

    cb = wid * NCH
    pltpu.sync_copy(cols_hbm.at[pl.ds(cb, NCH)], cols_all)

    def issue_gather(ch, b):
        pltpu.async_copy(x_hbm.at[cols_all.at[ch]], gbufs[b], gsems[b])

    def wait_gather(b):
        pltpu.make_async_copy(x_hbm.at[cols_all.at[0]], gbufs[b],
                              gsems[b]).wait()

    def issue_rv(ch, b):
        pltpu.async_copy(rows_hbm.at[cb + ch], rowbuf.at[b], rsems[b])
        pltpu.async_copy(vals_hbm.at[cb + ch], valbuf.at[b], vsems[b])

    def wait_rv(b):
        pltpu.make_async_copy(rows_hbm.at[cb], rowbuf.at[b],
                              rsems[b]).wait()
        pltpu.make_async_copy(vals_hbm.at[cb], valbuf.at[b],
                              vsems[b]).wait()

    def issue_scatter(b):
        pltpu.async_copy(gbufs[b], acc.at[rowbuf.at[b]], ssems[b],
                         add=True)

    def wait_scatter(b):
        pltpu.make_async_copy(gbufs[b], acc.at[rowbuf.at[b]],
                              ssems[b]).wait()

    issue_rv(0, 0)
    issue_gather(0, 0)

    zv = jnp.zeros((16,), jnp.float32)

    def zrow(r, carry):
        for j in range(8):
            gb1[r, pl.ds(j * 16, 16)] = zv
        return carry

    lax.fori_loop(0, CH, zrow, 0)
    r0 = s * RPT
    for k in range(RPT // CH):
        pltpu.sync_copy(gb1, acc.at[pl.ds(r0 + k * CH, CH)])
    plsc.subcore_barrier()

    def pair(i2, carry):
        for b in range(2):
            ch = i2 * 2 + b
            nb = 1 - b
            wait_rv(b)

            @pl.when(ch >= 1)
            def _():
                wait_scatter(nb)

            @pl.when(ch + 1 < NCH)
            def _():
                issue_rv(ch + 1, nb)
                issue_gather(ch + 1, nb)

            wait_gather(b)

            def scale_group(g, carry2):
                vval = valbuf[b, pl.ds(g * 16, 16)]
                gbuf = gbufs[b]
                for k in range(16):
                    e = g * 16 + k
                    vv = _lane_bcast(vval, k)
                    for j in range(8):
                        sl = pl.ds(j * 16, 16)
                        gbuf[e, sl] = gbuf[e, sl] * vv
                return carry2

            lax.fori_loop(0, CH // 16, scale_group, 0)
            issue_scatter(b)
        return carry

    lax.fori_loop(0, NCH // 2, pair, 0)
    wait_scatter(1)
    plsc.subcore_barrier()

    for k in range(RPT // CH):
        sl = pl.ds(r0 + k * CH, CH)
        pltpu.sync_copy(acc.at[sl], gb0)
        pltpu.sync_copy(gb0, out_hbm.at[c, sl])


def _spmm_call(rows2, vals2, cols2, x):
    mesh = plsc.VectorSubcoreMesh(core_axis_name="c", subcore_axis_name="s")
    fn = pl.kernel(
        _spmm_sc,
        out_type=jax.ShapeDtypeStruct((NC, NPAD, D), jnp.float32),
        mesh=mesh,
        scratch_types=[
            pltpu.VMEM_SHARED((NPAD, D), jnp.float32),
            pltpu.VMEM((NCH, CH), jnp.int32),
            pltpu.VMEM((2, CH), jnp.int32),
            pltpu.VMEM((2, CH), jnp.float32),
            pltpu.VMEM((CH, D), jnp.float32),
            pltpu.VMEM((CH, D), jnp.float32),
            pltpu.SemaphoreType.DMA,
            pltpu.SemaphoreType.DMA,
            pltpu.SemaphoreType.DMA,
            pltpu.SemaphoreType.DMA,
            pltpu.SemaphoreType.DMA,
            pltpu.SemaphoreType.DMA,
            pltpu.SemaphoreType.DMA,
            pltpu.SemaphoreType.DMA,
        ],
    )
    return fn(rows2, vals2, cols2, x)


def _gather_sc(seqmap_hbm, users_hbm, table_hbm, seq_out, sp_out,
               ubuf, idxbuf, gb0, gb1, sbuf, sem, gs0, gs1, ws0, ws1):
    c = lax.axis_index("c")
    s = lax.axis_index("s")
    wid = s * NC + c
    ub = wid * UPW
    gbufs = (gb0, gb1)
    gsems = (gs0, gs1)
    wsems = (ws0, ws1)
    pltpu.sync_copy(users_hbm.at[pl.ds(ub, UPW)], ubuf)
    pltpu.async_copy(seqmap_hbm.at[ubuf], idxbuf, sem).wait()

    def issue_user(u, b):
        pltpu.async_copy(table_hbm.at[idxbuf.at[u, pl.ds(0, T)]], gbufs[b],
                         gsems[b])

    def wait_user(b):
        pltpu.make_async_copy(table_hbm.at[idxbuf.at[0, pl.ds(0, T)]],
                              gbufs[b], gsems[b]).wait()

    def issue_write(u, b):
        pltpu.async_copy(gbufs[b], seq_out.at[ub + u], wsems[b])

    def wait_write(b):
        pltpu.make_async_copy(gbufs[b], seq_out.at[ub], wsems[b]).wait()

    issue_user(0, 0)
    pltpu.async_copy(table_hbm.at[ubuf], sbuf, sem).wait()
    pltpu.sync_copy(sbuf, sp_out.at[pl.ds(ub, UPW)])

    def pair(i2, carry):
        for b in range(2):
            u = i2 * 2 + b
            nb = 1 - b

            @pl.when(u >= 1)
            def _():
                wait_write(nb)

            @pl.when(u + 1 < UPW)
            def _():
                issue_user(u + 1, nb)

            wait_user(b)
            issue_write(u, b)
        return carry

    lax.fori_loop(0, UPW // 2, pair, 0)
    wait_write(1)


def _gather_call(seqmap_pad, users, ue_ie):
    mesh = plsc.VectorSubcoreMesh(core_axis_name="c", subcore_axis_name="s")
    fn = pl.kernel(
        _gather_sc,
        out_type=(
            jax.ShapeDtypeStruct((B, T, D), jnp.float32),
            jax.ShapeDtypeStruct((B, D), jnp.float32),
        ),
        mesh=mesh,
        scratch_types=[
            pltpu.VMEM((UPW,), jnp.int32),
            pltpu.VMEM((UPW, TPADC), jnp.int32),
            pltpu.VMEM((T, D), jnp.float32),
            pltpu.VMEM((T, D), jnp.float32),
            pltpu.VMEM((UPW, D), jnp.float32),
            pltpu.SemaphoreType.DMA,
            pltpu.SemaphoreType.DMA,
            pltpu.SemaphoreType.DMA,
            pltpu.SemaphoreType.DMA,
            pltpu.SemaphoreType.DMA,
        ],
    )
    return fn(seqmap_pad, users, ue_ie)


_CBLK = 1000


def _combine1_body(x_ref, p_ref, h1_ref, f1_ref):
    h1 = p_ref[0] + p_ref[1]
    h1_ref[...] = h1
    f1_ref[...] = x_ref[...] + h1


def _combine1(x, p):
    grid = N // _CBLK
    return pl.pallas_call(
        _combine1_body,
        grid=(grid,),
        in_specs=[
            pl.BlockSpec((_CBLK, D), lambda i: (i, 0)),
            pl.BlockSpec((NC, _CBLK, D), lambda i: (0, i, 0)),
        ],
        out_specs=[
            pl.BlockSpec((_CBLK, D), lambda i: (i, 0)),
            pl.BlockSpec((_CBLK, D), lambda i: (i, 0)),
        ],
        out_shape=[
            jax.ShapeDtypeStruct((N, D), jnp.float32),
            jax.ShapeDtypeStruct((N, D), jnp.float32),
        ],
    )(x, p)


def _combine2_body(f1_ref, q_ref, out_ref):
    f = f1_ref[...] + q_ref[0] + q_ref[1]
    nrm = jnp.sqrt(jnp.sum(f * f, axis=1, keepdims=True))
    out_ref[...] = f / jnp.maximum(nrm, 1e-12)


def _combine2(f1, q):
    grid = N // _CBLK
    return pl.pallas_call(
        _combine2_body,
        grid=(grid,),
        in_specs=[
            pl.BlockSpec((_CBLK, D), lambda i: (i, 0)),
            pl.BlockSpec((NC, _CBLK, D), lambda i: (0, i, 0)),
        ],
        out_specs=pl.BlockSpec((_CBLK, D), lambda i: (i, 0)),
        out_shape=jax.ShapeDtypeStruct((N, D), jnp.float32),
    )(f1, q)


def _lstm_body(seq_ref, w_ref, b_ref, out_ref, h_ref, c_ref):
    t = pl.program_id(0)

    @pl.when(t == 0)
    def _():
        h_ref[...] = jnp.zeros_like(h_ref)
        c_ref[...] = jnp.zeros_like(c_ref)

    xt = seq_ref[...]
    h = h_ref[...]
    cc = c_ref[...]
    w = w_ref[...]
    z = (jnp.dot(xt.astype(jnp.bfloat16), w[:D],
                 preferred_element_type=jnp.float32)
         + jnp.dot(h.astype(jnp.bfloat16), w[D:],
                   preferred_element_type=jnp.float32)
         + b_ref[...])
    ig = jax.nn.sigmoid(z[:, :D])
    fg = jax.nn.sigmoid(z[:, D:2 * D])
    gg = jnp.tanh(z[:, 2 * D:3 * D])
    og = jax.nn.sigmoid(z[:, 3 * D:])
    cc = fg * cc + ig * gg
    h = og * jnp.tanh(cc)
    h_ref[...] = h
    c_ref[...] = cc

    @pl.when(t == T - 1)
    def _():
        out_ref[...] = h


def _lstm_call(seq2, Wc, bcomb):
    return pl.pallas_call(
        _lstm_body,
        grid=(T,),
        in_specs=[
            pl.BlockSpec((B, D), lambda t: (0, t)),
            pl.BlockSpec((2 * D, 4 * D), lambda t: (0, 0)),
            pl.BlockSpec((1, 4 * D), lambda t: (0, 0)),
        ],
        out_specs=pl.BlockSpec((B, D), lambda t: (0, 0)),
        out_shape=jax.ShapeDtypeStruct((B, D), jnp.float32),
        scratch_shapes=[
            pltpu.VMEM((B, D), jnp.float32),
            pltpu.VMEM((B, D), jnp.float32),
        ],
    )(seq2, Wc, bcomb)


def _fuse_body(sp_ref, tm_ref, aw_ref, ab_ref, out_ref):
    sp = sp_ref[...]
    tm = tm_ref[...]
    aw = aw_ref[...]
    ab = ab_ref[...]
    l0 = (jnp.sum(sp * aw[0, :D][None, :], axis=1)
          + jnp.sum(tm * aw[0, D:][None, :], axis=1) + ab[0, 0])
    l1 = (jnp.sum(sp * aw[1, :D][None, :], axis=1)
          + jnp.sum(tm * aw[1, D:][None, :], axis=1) + ab[0, 1])
    m = jnp.maximum(l0, l1)
    e0 = jnp.exp(l0 - m)
    e1 = jnp.exp(l1 - m)
    ssum = e0 + e1
    w0 = e0 / ssum
    w1 = e1 / ssum
    out_ref[...] = w0[:, None] * sp + w1[:, None] * tm


def _fuse_call(spatial, temporal, attn_W, attn_b2):
    return pl.pallas_call(
        _fuse_body,
        out_shape=jax.ShapeDtypeStruct((B, D), jnp.float32),
    )(spatial, temporal, attn_W, attn_b2)


def kernel(adj_indices, adj_vals, user_seq_map, users, user_emb, item_emb,
           Wih, Whh, bih, bhh, attn_W, attn_b):
    rows = adj_indices[0].astype(jnp.int32)
    cols = adj_indices[1].astype(jnp.int32)
    pad = E_PAD - E
    pidx = jnp.arange(pad, dtype=jnp.int32) % N
    rows_p = jnp.concatenate([rows, pidx]).reshape(E_PAD // CH, CH)
    cols_p = jnp.concatenate([cols, pidx]).reshape(E_PAD // CH, CH)
    vals_p = jnp.concatenate([adj_vals, jnp.zeros((pad,), jnp.float32)]
                             ).reshape(E_PAD // CH, CH)
    x = jnp.concatenate([user_emb, item_emb], axis=0)

    p = _spmm_call(rows_p, vals_p, cols_p, x)
    h1, f1 = _combine1(x, p)
    q = _spmm_call(rows_p, vals_p, cols_p, h1)
    ue_ie = _combine2(f1, q)
    ie = ue_ie[U:]

    seqmap_pad = jnp.pad(user_seq_map.astype(jnp.int32) + U,
                         ((0, 0), (0, TPADC - T)))
    seq_flat, spatial = _gather_call(seqmap_pad, users.astype(jnp.int32),
                                     ue_ie)
    seq2 = seq_flat.reshape(B, T * D)

    Wc = jnp.concatenate([Wih.T, Whh.T], axis=0).astype(jnp.bfloat16)
    bcomb = (bih + bhh).reshape(1, 4 * D)
    temporal = _lstm_call(seq2, Wc, bcomb)
    fused = _fuse_call(spatial, temporal, attn_W, attn_b.reshape(1, 2))
    return (fused, ie)


# speedup vs baseline: 9.6084x; 1.1285x over previous
import jax
import jax.numpy as jnp
from jax import lax
from jax.experimental import pallas as pl
from jax.experimental.pallas import tpu as pltpu
from jax.experimental.pallas import tpu_sc as plsc

U = 5000
I = 5000
N = U + I
E = 320000
D = 128
T = 50
B = 1024

NC = 2
NS = 16
NW = NC * NS
EPW = 10240
E_PAD = NW * EPW
CH = 128
NCH = EPW // CH
NPAD = 10240
RPT = NPAD // NS
UPW = B // NW
TPADC = 128


def _lane_bcast(v, k):
    idx = jnp.full((16, 1), k, jnp.int32)
    dn = lax.GatherDimensionNumbers(offset_dims=(), collapsed_slice_dims=(0,),
                                    start_index_map=(0,))
    return lax.gather(v, idx, dn, (1,),
                      mode=lax.GatherScatterMode.PROMISE_IN_BOUNDS)


def _spmm_sc(rows_hbm, vals_hbm, cols_hbm, x_hbm, out_hbm,
             acc, cols_all, rowbuf, valbuf,
             gb0, gb1, gs0, gs1, ss0, ss1, rs0, rs1, vs0, vs1):
    c = lax.axis_index("c")
    s = lax.axis_index("s")
    wid = s * NC + c
    gbufs = (gb0, gb1)
    gsems = (gs0, gs1)
    ssems = (ss0, ss1)
    rsems = (rs0, rs1)
    vsems = (vs0, vs1)

    cb = wid * NCH
    pltpu.sync_copy(cols_hbm.at[pl.ds(cb, NCH)], cols_all)

    def issue_gather(ch, b):
        pltpu.async_copy(x_hbm.at[cols_all.at[ch]], gbufs[b], gsems[b])

    def wait_gather(b):
        pltpu.make_async_copy(x_hbm.at[cols_all.at[0]], gbufs[b],
                              gsems[b]).wait()

    def issue_rv(ch, b):
        pltpu.async_copy(rows_hbm.at[cb + ch], rowbuf.at[b], rsems[b])
        pltpu.async_copy(vals_hbm.at[cb + ch], valbuf.at[b], vsems[b])

    def wait_rv(b):
        pltpu.make_async_copy(rows_hbm.at[cb], rowbuf.at[b],
                              rsems[b]).wait()
        pltpu.make_async_copy(vals_hbm.at[cb], valbuf.at[b],
                              vsems[b]).wait()

    def issue_scatter(b):
        pltpu.async_copy(gbufs[b], acc.at[rowbuf.at[b]], ssems[b],
                         add=True)

    def wait_scatter(b):
        pltpu.make_async_copy(gbufs[b], acc.at[rowbuf.at[b]],
                              ssems[b]).wait()

    issue_rv(0, 0)
    issue_gather(0, 0)

    zv = jnp.zeros((16,), jnp.float32)

    def zrow(r, carry):
        for j in range(8):
            gb1[r, pl.ds(j * 16, 16)] = zv
        return carry

    lax.fori_loop(0, CH, zrow, 0)
    r0 = s * RPT
    for k in range(RPT // CH):
        pltpu.sync_copy(gb1, acc.at[pl.ds(r0 + k * CH, CH)])
    plsc.subcore_barrier()

    def pair(i2, carry):
        for b in range(2):
            ch = i2 * 2 + b
            nb = 1 - b
            wait_rv(b)

            @pl.when(ch >= 1)
            def _():
                wait_scatter(nb)

            @pl.when(ch + 1 < NCH)
            def _():
                issue_rv(ch + 1, nb)
                issue_gather(ch + 1, nb)

            wait_gather(b)

            def scale_group(g, carry2):
                vval = valbuf[b, pl.ds(g * 16, 16)]
                gbuf = gbufs[b]
                for k in range(16):
                    e = g * 16 + k
                    vv = _lane_bcast(vval, k)
                    for j in range(8):
                        sl = pl.ds(j * 16, 16)
                        gbuf[e, sl] = gbuf[e, sl] * vv
                return carry2

            lax.fori_loop(0, CH // 16, scale_group, 0)
            issue_scatter(b)
        return carry

    lax.fori_loop(0, NCH // 2, pair, 0)
    wait_scatter(1)
    plsc.subcore_barrier()

    for k in range(RPT // CH):
        sl = pl.ds(r0 + k * CH, CH)
        pltpu.sync_copy(acc.at[sl], gb0)
        pltpu.sync_copy(gb0, out_hbm.at[c, sl])


def _spmm_call(rows2, vals2, cols2, x):
    mesh = plsc.VectorSubcoreMesh(core_axis_name="c", subcore_axis_name="s")
    fn = pl.kernel(
        _spmm_sc,
        out_type=jax.ShapeDtypeStruct((NC, NPAD, D), jnp.float32),
        mesh=mesh,
        scratch_types=[
            pltpu.VMEM_SHARED((NPAD, D), jnp.float32),
            pltpu.VMEM((NCH, CH), jnp.int32),
            pltpu.VMEM((2, CH), jnp.int32),
            pltpu.VMEM((2, CH), jnp.float32),
            pltpu.VMEM((CH, D), jnp.float32),
            pltpu.VMEM((CH, D), jnp.float32),
            pltpu.SemaphoreType.DMA,
            pltpu.SemaphoreType.DMA,
            pltpu.SemaphoreType.DMA,
            pltpu.SemaphoreType.DMA,
            pltpu.SemaphoreType.DMA,
            pltpu.SemaphoreType.DMA,
            pltpu.SemaphoreType.DMA,
            pltpu.SemaphoreType.DMA,
        ],
    )
    return fn(rows2, vals2, cols2, x)


def _gather_sc(seqmap_hbm, users_hbm, table_hbm, seq_out, sp_out,
               ubuf, idxbuf, tix0, tix1, gb0, gb1, sbuf,
               sem, gs0, gs1, ws0, ws1):
    c = lax.axis_index("c")
    s = lax.axis_index("s")
    wid = s * NC + c
    ub = wid * UPW
    gbufs = (gb0, gb1)
    gsems = (gs0, gs1)
    wsems = (ws0, ws1)
    tixs = (tix0, tix1)
    pltpu.sync_copy(users_hbm.at[pl.ds(ub, UPW)], ubuf)
    pltpu.async_copy(seqmap_hbm.at[ubuf], idxbuf, sem).wait()

    iota16 = jnp.arange(16, dtype=jnp.int32)

    def issue_user(u, b):
        pltpu.async_copy(table_hbm.at[idxbuf.at[u, pl.ds(0, T)]], gbufs[b],
                         gsems[b])

    def wait_user(b):
        pltpu.make_async_copy(table_hbm.at[idxbuf.at[0, pl.ds(0, T)]],
                              gbufs[b], gsems[b]).wait()

    def issue_write(u, b):
        ug = ub + u
        for off in (0, 16, 32, T - 16):
            tixs[b][pl.ds(off, 16)] = (iota16 + off) * B + ug
        pltpu.async_copy(gbufs[b], seq_out.at[tixs[b]], wsems[b])

    def wait_write(b):
        pltpu.make_async_copy(gbufs[b], seq_out.at[tixs[b]],
                              wsems[b]).wait()

    issue_user(0, 0)
    pltpu.async_copy(table_hbm.at[ubuf], sbuf, sem).wait()
    pltpu.sync_copy(sbuf, sp_out.at[pl.ds(ub, UPW)])

    def pair(i2, carry):
        for b in range(2):
            u = i2 * 2 + b
            nb = 1 - b

            @pl.when(u >= 1)
            def _():
                wait_write(nb)

            @pl.when(u + 1 < UPW)
            def _():
                issue_user(u + 1, nb)

            wait_user(b)
            issue_write(u, b)
        return carry

    lax.fori_loop(0, UPW // 2, pair, 0)
    wait_write(1)


def _gather_call(seqmap_pad, users, ue_ie):
    mesh = plsc.VectorSubcoreMesh(core_axis_name="c", subcore_axis_name="s")
    fn = pl.kernel(
        _gather_sc,
        out_type=(
            jax.ShapeDtypeStruct((T * B, D), jnp.float32),
            jax.ShapeDtypeStruct((B, D), jnp.float32),
        ),
        mesh=mesh,
        scratch_types=[
            pltpu.VMEM((UPW,), jnp.int32),
            pltpu.VMEM((UPW, TPADC), jnp.int32),
            pltpu.VMEM((T,), jnp.int32),
            pltpu.VMEM((T,), jnp.int32),
            pltpu.VMEM((T, D), jnp.float32),
            pltpu.VMEM((T, D), jnp.float32),
            pltpu.VMEM((UPW, D), jnp.float32),
            pltpu.SemaphoreType.DMA,
            pltpu.SemaphoreType.DMA,
            pltpu.SemaphoreType.DMA,
            pltpu.SemaphoreType.DMA,
            pltpu.SemaphoreType.DMA,
        ],
    )
    return fn(seqmap_pad, users, ue_ie)


_CBLK = 1000


def _combine1_body(x_ref, p_ref, h1_ref, f1_ref):
    h1 = p_ref[0] + p_ref[1]
    h1_ref[...] = h1
    f1_ref[...] = x_ref[...] + h1


def _combine1(x, p):
    grid = N // _CBLK
    return pl.pallas_call(
        _combine1_body,
        grid=(grid,),
        in_specs=[
            pl.BlockSpec((_CBLK, D), lambda i: (i, 0)),
            pl.BlockSpec((NC, _CBLK, D), lambda i: (0, i, 0)),
        ],
        out_specs=[
            pl.BlockSpec((_CBLK, D), lambda i: (i, 0)),
            pl.BlockSpec((_CBLK, D), lambda i: (i, 0)),
        ],
        out_shape=[
            jax.ShapeDtypeStruct((N, D), jnp.float32),
            jax.ShapeDtypeStruct((N, D), jnp.float32),
        ],
    )(x, p)


def _combine2_body(f1_ref, q_ref, out_ref):
    f = f1_ref[...] + q_ref[0] + q_ref[1]
    nrm = jnp.sqrt(jnp.sum(f * f, axis=1, keepdims=True))
    out_ref[...] = f / jnp.maximum(nrm, 1e-12)


def _combine2(f1, q):
    grid = N // _CBLK
    return pl.pallas_call(
        _combine2_body,
        grid=(grid,),
        in_specs=[
            pl.BlockSpec((_CBLK, D), lambda i: (i, 0)),
            pl.BlockSpec((NC, _CBLK, D), lambda i: (0, i, 0)),
        ],
        out_specs=pl.BlockSpec((_CBLK, D), lambda i: (i, 0)),
        out_shape=jax.ShapeDtypeStruct((N, D), jnp.float32),
    )(f1, q)


def _lstm_body(seq_ref, w_ref, b_ref, out_ref, h_ref, c_ref):
    t = pl.program_id(0)

    @pl.when(t == 0)
    def _():
        h_ref[...] = jnp.zeros_like(h_ref)
        c_ref[...] = jnp.zeros_like(c_ref)

    xt = seq_ref[0]
    h = h_ref[...]
    cc = c_ref[...]
    w = w_ref[...]
    z = (jnp.dot(xt.astype(jnp.bfloat16), w[:D],
                 preferred_element_type=jnp.float32)
         + jnp.dot(h.astype(jnp.bfloat16), w[D:],
                   preferred_element_type=jnp.float32)
         + b_ref[...])
    ig = jax.nn.sigmoid(z[:, :D])
    fg = jax.nn.sigmoid(z[:, D:2 * D])
    gg = jnp.tanh(z[:, 2 * D:3 * D])
    og = jax.nn.sigmoid(z[:, 3 * D:])
    cc = fg * cc + ig * gg
    h = og * jnp.tanh(cc)
    h_ref[...] = h
    c_ref[...] = cc

    @pl.when(t == T - 1)
    def _():
        out_ref[...] = h


def _lstm_call(seq3, Wc, bcomb):
    return pl.pallas_call(
        _lstm_body,
        grid=(T,),
        in_specs=[
            pl.BlockSpec((1, B, D), lambda t: (t, 0, 0)),
            pl.BlockSpec((2 * D, 4 * D), lambda t: (0, 0)),
            pl.BlockSpec((1, 4 * D), lambda t: (0, 0)),
        ],
        out_specs=pl.BlockSpec((B, D), lambda t: (0, 0)),
        out_shape=jax.ShapeDtypeStruct((B, D), jnp.float32),
        scratch_shapes=[
            pltpu.VMEM((B, D), jnp.float32),
            pltpu.VMEM((B, D), jnp.float32),
        ],
    )(seq3, Wc, bcomb)


def _fuse_body(sp_ref, tm_ref, aw_ref, ab_ref, out_ref):
    sp = sp_ref[...]
    tm = tm_ref[...]
    aw = aw_ref[...]
    ab = ab_ref[...]
    l0 = (jnp.sum(sp * aw[0, :D][None, :], axis=1)
          + jnp.sum(tm * aw[0, D:][None, :], axis=1) + ab[0, 0])
    l1 = (jnp.sum(sp * aw[1, :D][None, :], axis=1)
          + jnp.sum(tm * aw[1, D:][None, :], axis=1) + ab[0, 1])
    m = jnp.maximum(l0, l1)
    e0 = jnp.exp(l0 - m)
    e1 = jnp.exp(l1 - m)
    ssum = e0 + e1
    w0 = e0 / ssum
    w1 = e1 / ssum
    out_ref[...] = w0[:, None] * sp + w1[:, None] * tm


def _fuse_call(spatial, temporal, attn_W, attn_b2):
    return pl.pallas_call(
        _fuse_body,
        out_shape=jax.ShapeDtypeStruct((B, D), jnp.float32),
    )(spatial, temporal, attn_W, attn_b2)


def kernel(adj_indices, adj_vals, user_seq_map, users, user_emb, item_emb,
           Wih, Whh, bih, bhh, attn_W, attn_b):
    rows = adj_indices[0].astype(jnp.int32)
    cols = adj_indices[1].astype(jnp.int32)
    pad = E_PAD - E
    pidx = jnp.arange(pad, dtype=jnp.int32) % N
    rows_p = jnp.concatenate([rows, pidx]).reshape(E_PAD // CH, CH)
    cols_p = jnp.concatenate([cols, pidx]).reshape(E_PAD // CH, CH)
    vals_p = jnp.concatenate([adj_vals, jnp.zeros((pad,), jnp.float32)]
                             ).reshape(E_PAD // CH, CH)
    x = jnp.concatenate([user_emb, item_emb], axis=0)

    p = _spmm_call(rows_p, vals_p, cols_p, x)
    h1, f1 = _combine1(x, p)
    q = _spmm_call(rows_p, vals_p, cols_p, h1)
    ue_ie = _combine2(f1, q)
    ie = ue_ie[U:]

    seqmap_pad = jnp.pad(user_seq_map.astype(jnp.int32) + U,
                         ((0, 0), (0, TPADC - T)))
    seq_flat, spatial = _gather_call(seqmap_pad, users.astype(jnp.int32),
                                     ue_ie)
    seq3 = seq_flat.reshape(T, B, D)

    Wc = jnp.concatenate([Wih.T, Whh.T], axis=0).astype(jnp.bfloat16)
    bcomb = (bih + bhh).reshape(1, 4 * D)
    temporal = _lstm_call(seq3, Wc, bcomb)
    fused = _fuse_call(spatial, temporal, attn_W, attn_b.reshape(1, 2))
    return (fused, ie)
